# Initial kernel scaffold; baseline (speedup 1.0000x reference)
import jax
import jax.numpy as jnp
from jax.experimental import pallas as pl


def kernel(x, edge_index, graph_ids, x_adduct, W_proj, b_proj, W_gin, b_gin, eps, W1, b1, W2, b2, Wo, bo):
    raise NotImplementedError("write your pallas kernel here")


# speedup vs baseline: 4.8299x; 4.8299x over previous
import functools

import jax
import jax.numpy as jnp
from jax import lax
from jax.experimental import pallas as pl
from jax.experimental.pallas import tpu as pltpu
from jax.experimental.pallas import tpu_sc as plsc

N_NODES = 10000
N_EDGES = 320000
D = 128
N_GRAPHS = 256
NC = 2
NS = 16
NW = NC * NS
EPW = N_EDGES // NW
CH = 80
NCHUNK = EPW // CH
NODE_CHUNKS = N_NODES // CH

def _sc_agg_readout_body(h_hbm, src_hbm, dst_hbm, gid_hbm, zeros_hbm,
                         agg_out, g_out,
                         idx_a, idx_b, rows, sem, acc, gacc):
    cid = lax.axis_index("c")
    sid = lax.axis_index("s")
    wid = sid * NC + cid

    @pl.when(sid == 0)
    def _init():
        pltpu.sync_copy(zeros_hbm, acc)
        pltpu.sync_copy(zeros_hbm.at[pl.ds(0, N_GRAPHS)], gacc)

    plsc.subcore_barrier()

    base = wid * EPW

    def estep(c, carry):
        off = base + c * CH
        pltpu.sync_copy(src_hbm.at[pl.ds(off, CH)], idx_a)
        pltpu.sync_copy(dst_hbm.at[pl.ds(off, CH)], idx_b)
        pltpu.async_copy(h_hbm.at[idx_a], rows, sem).wait()
        pltpu.sync_copy(rows, acc.at[idx_b], add=True)
        return carry

    lax.fori_loop(0, NCHUNK, estep, 0)

    def rstep(k, carry):
        c = wid + NW * k

        @pl.when(c < NODE_CHUNKS)
        def _():
            off = c * CH
            pltpu.sync_copy(h_hbm.at[pl.ds(off, CH)], rows)
            pltpu.sync_copy(gid_hbm.at[pl.ds(off, CH)], idx_a)
            pltpu.sync_copy(rows, gacc.at[idx_a], add=True)

        return carry

    lax.fori_loop(0, (NODE_CHUNKS + NW - 1) // NW, rstep, 0)

    plsc.subcore_barrier()

    @pl.when(sid == 0)
    def _writeout():
        pltpu.sync_copy(acc, agg_out.at[cid])
        pltpu.sync_copy(gacc, g_out.at[cid])


@functools.cache
def _sc_agg_readout_kernel():
    mesh = plsc.VectorSubcoreMesh(core_axis_name="c", subcore_axis_name="s")
    return functools.partial(
        pl.kernel,
        out_type=(
            jax.ShapeDtypeStruct((NC, N_NODES, D), jnp.float32),
            jax.ShapeDtypeStruct((NC, N_GRAPHS, D), jnp.float32),
        ),
        mesh=mesh,
        scratch_types=[
            pltpu.VMEM((CH,), jnp.int32),
            pltpu.VMEM((CH,), jnp.int32),
            pltpu.VMEM((CH, D), jnp.float32),
            pltpu.SemaphoreType.DMA,
            pltpu.VMEM_SHARED((N_NODES, D), jnp.float32),
            pltpu.VMEM_SHARED((N_GRAPHS, D), jnp.float32),
        ],
    )(_sc_agg_readout_body)


def _sc_agg_readout(h, src, dst, gid, zeros):
    return _sc_agg_readout_kernel()(h, src, dst, gid, zeros)


def _sc_readout_body(h_hbm, gid_hbm, zeros_hbm, g_out,
                     idx_a, rows, gacc):
    cid = lax.axis_index("c")
    sid = lax.axis_index("s")
    wid = sid * NC + cid

    @pl.when(sid == 0)
    def _init():
        pltpu.sync_copy(zeros_hbm.at[pl.ds(0, N_GRAPHS)], gacc)

    plsc.subcore_barrier()

    def rstep(k, carry):
        c = wid + NW * k

        @pl.when(c < NODE_CHUNKS)
        def _():
            off = c * CH
            pltpu.sync_copy(h_hbm.at[pl.ds(off, CH)], rows)
            pltpu.sync_copy(gid_hbm.at[pl.ds(off, CH)], idx_a)
            pltpu.sync_copy(rows, gacc.at[idx_a], add=True)

        return carry

    lax.fori_loop(0, (NODE_CHUNKS + NW - 1) // NW, rstep, 0)

    plsc.subcore_barrier()

    @pl.when(sid == 0)
    def _writeout():
        pltpu.sync_copy(gacc, g_out.at[cid])


@functools.cache
def _sc_readout_kernel():
    mesh = plsc.VectorSubcoreMesh(core_axis_name="c", subcore_axis_name="s")
    return functools.partial(
        pl.kernel,
        out_type=jax.ShapeDtypeStruct((NC, N_GRAPHS, D), jnp.float32),
        mesh=mesh,
        scratch_types=[
            pltpu.VMEM((CH,), jnp.int32),
            pltpu.VMEM((CH, D), jnp.float32),
            pltpu.VMEM_SHARED((N_GRAPHS, D), jnp.float32),
        ],
    )(_sc_readout_body)


def _sc_readout(h, gid, zeros):
    return _sc_readout_kernel()(h, gid, zeros)


_ROWS_BLK = 1000
_GRID = N_NODES // _ROWS_BLK


def _proj_body(x_ref, w_ref, b_ref, o_ref):
    o_ref[...] = (
        jnp.dot(x_ref[...], w_ref[...], preferred_element_type=jnp.float32)
        + b_ref[...]
    )


def _proj(x, W, b):
    return pl.pallas_call(
        _proj_body,
        grid=(_GRID,),
        in_specs=[
            pl.BlockSpec((_ROWS_BLK, D), lambda i: (i, 0)),
            pl.BlockSpec((D, D), lambda i: (0, 0)),
            pl.BlockSpec((1, D), lambda i: (0, 0)),
        ],
        out_specs=pl.BlockSpec((_ROWS_BLK, D), lambda i: (i, 0)),
        out_shape=jax.ShapeDtypeStruct((N_NODES, D), jnp.float32),
    )(x, W, b)


def _gin_body(h_ref, a0_ref, a1_ref, w_ref, b_ref, s_ref, o_ref):
    t = s_ref[...] * h_ref[...] + a0_ref[...] + a1_ref[...]
    o_ref[...] = jnp.maximum(
        jnp.dot(t, w_ref[...], preferred_element_type=jnp.float32) + b_ref[...],
        0.0,
    )


def _gin_combine(h, a0, a1, W, b, scale):
    return pl.pallas_call(
        _gin_body,
        grid=(_GRID,),
        in_specs=[
            pl.BlockSpec((_ROWS_BLK, D), lambda i: (i, 0)),
            pl.BlockSpec((_ROWS_BLK, D), lambda i: (i, 0)),
            pl.BlockSpec((_ROWS_BLK, D), lambda i: (i, 0)),
            pl.BlockSpec((D, D), lambda i: (0, 0)),
            pl.BlockSpec((1, D), lambda i: (0, 0)),
            pl.BlockSpec((1, D), lambda i: (0, 0)),
        ],
        out_specs=pl.BlockSpec((_ROWS_BLK, D), lambda i: (i, 0)),
        out_shape=jax.ShapeDtypeStruct((N_NODES, D), jnp.float32),
    )(h, a0, a1, W, b, scale)


def _head_body(g_ref, a_ref, w1g_ref, w1a_ref, b1_ref, w2_ref, b2_ref,
               wo_ref, bo_ref, o_ref):
    acc = (
        jnp.dot(a_ref[...], w1a_ref[...], preferred_element_type=jnp.float32)
        + b1_ref[...]
    )
    for i in range(4):
        gi = g_ref[i, 0] + g_ref[i, 1]
        acc = acc + jnp.dot(
            gi, w1g_ref[i * D:(i + 1) * D, :], preferred_element_type=jnp.float32
        )
    z = jnp.maximum(acc, 0.0)
    z = jnp.maximum(
        jnp.dot(z, w2_ref[...], preferred_element_type=jnp.float32) + b2_ref[...],
        0.0,
    )
    o_ref[...] = (
        jnp.dot(z, wo_ref[...], preferred_element_type=jnp.float32) + bo_ref[...]
    )


def _head(G, A, W1g, W1a, b1, W2, b2, Wo, bo):
    return pl.pallas_call(
        _head_body,
        out_shape=jax.ShapeDtypeStruct((N_GRAPHS, D), jnp.float32),
    )(G, A, W1g, W1a, b1, W2, b2, Wo, bo)


def kernel(x, edge_index, graph_ids, x_adduct, W_proj, b_proj, W_gin, b_gin,
           eps, W1, b1, W2, b2, Wo, bo):
    src = edge_index[0]
    dst = edge_index[1]
    zeros = jnp.zeros((N_NODES, D), jnp.float32)

    h = _proj(x, W_proj, b_proj.reshape(1, D))
    g_parts = []
    for i in range(3):
        agg_p, g_p = _sc_agg_readout(h, src, dst, graph_ids, zeros)
        g_parts.append(g_p)
        scale = jnp.full((1, D), 1.0 + eps[i], jnp.float32)
        h = _gin_combine(h, agg_p[0], agg_p[1], W_gin[i],
                         b_gin[i].reshape(1, D), scale)
    g_parts.append(_sc_readout(h, graph_ids, zeros))

    G = jnp.stack(g_parts)
    A = jnp.pad(x_adduct, ((0, 0), (0, D - 8)))
    W1g = W1[: 4 * D]
    W1a = jnp.pad(W1[4 * D:], ((0, D - 8), (0, 0)))
    Wop = jnp.pad(Wo, ((0, 0), (0, D - 1)))
    bop = jnp.pad(bo, (0, D - 1)).reshape(1, D)

    out = _head(G, A, W1g, W1a, b1.reshape(1, -1), W2, b2.reshape(1, -1),
                Wop, bop)
    return out[:, :1]


# speedup vs baseline: 8.8945x; 1.8415x over previous
import functools

import jax
import jax.numpy as jnp
from jax import lax
from jax.experimental import pallas as pl
from jax.experimental.pallas import tpu as pltpu
from jax.experimental.pallas import tpu_sc as plsc

N_NODES = 10000
N_EDGES = 320000
D = 128
N_GRAPHS = 256
NC = 2
NS = 16
NW = NC * NS
EPW = N_EDGES // NW
CH = 80
NCHUNK = EPW // CH
NODE_CHUNKS = N_NODES // CH

def _sc_agg_readout_body(h_hbm, src_hbm, dst_hbm, gid_hbm, zeros_hbm,
                         agg_out, g_out,
                         idx_s0, idx_s1, idx_d0, idx_d1, rows0, rows1,
                         seml0, seml1, semg0, semg1, sems0, sems1,
                         acc, gacc):
    cid = lax.axis_index("c")
    sid = lax.axis_index("s")
    wid = sid * NC + cid

    @pl.when(sid == 0)
    def _init():
        pltpu.sync_copy(zeros_hbm, acc)
        pltpu.sync_copy(zeros_hbm.at[pl.ds(0, N_GRAPHS)], gacc)

    plsc.subcore_barrier()

    base = wid * EPW
    idx_s = (idx_s0, idx_s1)
    idx_d = (idx_d0, idx_d1)
    rows = (rows0, rows1)
    seml = (seml0, seml1)
    semg = (semg0, semg1)
    sems = (sems0, sems1)

    def start_l(c, b):
        off = base + c * CH
        pltpu.async_copy(src_hbm.at[pl.ds(off, CH)], idx_s[b], seml[b])
        pltpu.async_copy(dst_hbm.at[pl.ds(off, CH)], idx_d[b], seml[b])

    def wait_l(b):
        pltpu.make_async_copy(src_hbm.at[pl.ds(0, CH)], idx_s[b], seml[b]).wait()
        pltpu.make_async_copy(dst_hbm.at[pl.ds(0, CH)], idx_d[b], seml[b]).wait()

    def start_g(b):
        pltpu.async_copy(h_hbm.at[idx_s[b]], rows[b], semg[b])

    def wait_g(b):
        pltpu.make_async_copy(h_hbm.at[idx_s[b]], rows[b], semg[b]).wait()

    def start_s(b):
        pltpu.async_copy(rows[b], acc.at[idx_d[b]], sems[b], add=True)

    def wait_s(b):
        pltpu.make_async_copy(rows[b], acc.at[idx_d[b]], sems[b]).wait()

    start_l(0, 0)
    start_l(1, 1)
    wait_l(0)
    start_g(0)

    def estep(i, carry):
        c0 = 2 * i
        for b in (0, 1):
            c = c0 + b
            nxt = c + 1

            @pl.when(nxt < NCHUNK)
            def _():
                wait_l(1 - b)
                start_g(1 - b)

            wait_g(b)
            start_s(b)
            wait_s(b)

            @pl.when(c + 2 < NCHUNK)
            def _():
                start_l(c + 2, b)

        return carry

    lax.fori_loop(0, (NCHUNK - 1) // 2, estep, 0, unroll=False)
    b_last = (NCHUNK - 1) % 2
    wait_g(b_last)
    start_s(b_last)
    wait_s(b_last)

    def rstep(k, carry):
        c = wid + NW * k

        @pl.when(c < NODE_CHUNKS)
        def _():
            off = c * CH
            pltpu.sync_copy(h_hbm.at[pl.ds(off, CH)], rows0)
            pltpu.sync_copy(gid_hbm.at[pl.ds(off, CH)], idx_s0)
            pltpu.sync_copy(rows0, gacc.at[idx_s0], add=True)

        return carry

    lax.fori_loop(0, (NODE_CHUNKS + NW - 1) // NW, rstep, 0)

    plsc.subcore_barrier()

    @pl.when(sid == 0)
    def _writeout():
        pltpu.sync_copy(acc, agg_out.at[cid])
        pltpu.sync_copy(gacc, g_out.at[cid])


@functools.cache
def _sc_agg_readout_kernel():
    mesh = plsc.VectorSubcoreMesh(core_axis_name="c", subcore_axis_name="s")
    return functools.partial(
        pl.kernel,
        out_type=(
            jax.ShapeDtypeStruct((NC, N_NODES, D), jnp.float32),
            jax.ShapeDtypeStruct((NC, N_GRAPHS, D), jnp.float32),
        ),
        mesh=mesh,
        scratch_types=[
            pltpu.VMEM((CH,), jnp.int32),
            pltpu.VMEM((CH,), jnp.int32),
            pltpu.VMEM((CH,), jnp.int32),
            pltpu.VMEM((CH,), jnp.int32),
            pltpu.VMEM((CH, D), jnp.float32),
            pltpu.VMEM((CH, D), jnp.float32),
            pltpu.SemaphoreType.DMA,
            pltpu.SemaphoreType.DMA,
            pltpu.SemaphoreType.DMA,
            pltpu.SemaphoreType.DMA,
            pltpu.SemaphoreType.DMA,
            pltpu.SemaphoreType.DMA,
            pltpu.VMEM_SHARED((N_NODES, D), jnp.float32),
            pltpu.VMEM_SHARED((N_GRAPHS, D), jnp.float32),
        ],
    )(_sc_agg_readout_body)


def _sc_agg_readout(h, src, dst, gid, zeros):
    return _sc_agg_readout_kernel()(h, src, dst, gid, zeros)


def _sc_readout_body(h_hbm, gid_hbm, zeros_hbm, g_out,
                     idx_a, rows, gacc):
    cid = lax.axis_index("c")
    sid = lax.axis_index("s")
    wid = sid * NC + cid

    @pl.when(sid == 0)
    def _init():
        pltpu.sync_copy(zeros_hbm.at[pl.ds(0, N_GRAPHS)], gacc)

    plsc.subcore_barrier()

    def rstep(k, carry):
        c = wid + NW * k

        @pl.when(c < NODE_CHUNKS)
        def _():
            off = c * CH
            pltpu.sync_copy(h_hbm.at[pl.ds(off, CH)], rows)
            pltpu.sync_copy(gid_hbm.at[pl.ds(off, CH)], idx_a)
            pltpu.sync_copy(rows, gacc.at[idx_a], add=True)

        return carry

    lax.fori_loop(0, (NODE_CHUNKS + NW - 1) // NW, rstep, 0)

    plsc.subcore_barrier()

    @pl.when(sid == 0)
    def _writeout():
        pltpu.sync_copy(gacc, g_out.at[cid])


@functools.cache
def _sc_readout_kernel():
    mesh = plsc.VectorSubcoreMesh(core_axis_name="c", subcore_axis_name="s")
    return functools.partial(
        pl.kernel,
        out_type=jax.ShapeDtypeStruct((NC, N_GRAPHS, D), jnp.float32),
        mesh=mesh,
        scratch_types=[
            pltpu.VMEM((CH,), jnp.int32),
            pltpu.VMEM((CH, D), jnp.float32),
            pltpu.VMEM_SHARED((N_GRAPHS, D), jnp.float32),
        ],
    )(_sc_readout_body)


def _sc_readout(h, gid, zeros):
    return _sc_readout_kernel()(h, gid, zeros)


_ROWS_BLK = 1000
_GRID = N_NODES // _ROWS_BLK


def _proj_body(x_ref, w_ref, b_ref, o_ref):
    o_ref[...] = (
        jnp.dot(x_ref[...], w_ref[...], preferred_element_type=jnp.float32)
        + b_ref[...]
    )


def _proj(x, W, b):
    return pl.pallas_call(
        _proj_body,
        grid=(_GRID,),
        in_specs=[
            pl.BlockSpec((_ROWS_BLK, D), lambda i: (i, 0)),
            pl.BlockSpec((D, D), lambda i: (0, 0)),
            pl.BlockSpec((1, D), lambda i: (0, 0)),
        ],
        out_specs=pl.BlockSpec((_ROWS_BLK, D), lambda i: (i, 0)),
        out_shape=jax.ShapeDtypeStruct((N_NODES, D), jnp.float32),
    )(x, W, b)


def _gin_body(h_ref, a0_ref, a1_ref, w_ref, b_ref, s_ref, o_ref):
    t = s_ref[...] * h_ref[...] + a0_ref[...] + a1_ref[...]
    o_ref[...] = jnp.maximum(
        jnp.dot(t, w_ref[...], preferred_element_type=jnp.float32) + b_ref[...],
        0.0,
    )


def _gin_combine(h, a0, a1, W, b, scale):
    return pl.pallas_call(
        _gin_body,
        grid=(_GRID,),
        in_specs=[
            pl.BlockSpec((_ROWS_BLK, D), lambda i: (i, 0)),
            pl.BlockSpec((_ROWS_BLK, D), lambda i: (i, 0)),
            pl.BlockSpec((_ROWS_BLK, D), lambda i: (i, 0)),
            pl.BlockSpec((D, D), lambda i: (0, 0)),
            pl.BlockSpec((1, D), lambda i: (0, 0)),
            pl.BlockSpec((1, D), lambda i: (0, 0)),
        ],
        out_specs=pl.BlockSpec((_ROWS_BLK, D), lambda i: (i, 0)),
        out_shape=jax.ShapeDtypeStruct((N_NODES, D), jnp.float32),
    )(h, a0, a1, W, b, scale)


def _head_body(g_ref, a_ref, w1g_ref, w1a_ref, b1_ref, w2_ref, b2_ref,
               wo_ref, bo_ref, o_ref):
    acc = (
        jnp.dot(a_ref[...], w1a_ref[...], preferred_element_type=jnp.float32)
        + b1_ref[...]
    )
    for i in range(4):
        gi = g_ref[i, 0] + g_ref[i, 1]
        acc = acc + jnp.dot(
            gi, w1g_ref[i * D:(i + 1) * D, :], preferred_element_type=jnp.float32
        )
    z = jnp.maximum(acc, 0.0)
    z = jnp.maximum(
        jnp.dot(z, w2_ref[...], preferred_element_type=jnp.float32) + b2_ref[...],
        0.0,
    )
    o_ref[...] = (
        jnp.dot(z, wo_ref[...], preferred_element_type=jnp.float32) + bo_ref[...]
    )


def _head(G, A, W1g, W1a, b1, W2, b2, Wo, bo):
    return pl.pallas_call(
        _head_body,
        out_shape=jax.ShapeDtypeStruct((N_GRAPHS, D), jnp.float32),
    )(G, A, W1g, W1a, b1, W2, b2, Wo, bo)


def kernel(x, edge_index, graph_ids, x_adduct, W_proj, b_proj, W_gin, b_gin,
           eps, W1, b1, W2, b2, Wo, bo):
    src = edge_index[0]
    dst = edge_index[1]
    zeros = jnp.zeros((N_NODES, D), jnp.float32)

    h = _proj(x, W_proj, b_proj.reshape(1, D))
    g_parts = []
    for i in range(3):
        agg_p, g_p = _sc_agg_readout(h, src, dst, graph_ids, zeros)
        g_parts.append(g_p)
        scale = jnp.full((1, D), 1.0 + eps[i], jnp.float32)
        h = _gin_combine(h, agg_p[0], agg_p[1], W_gin[i],
                         b_gin[i].reshape(1, D), scale)
    g_parts.append(_sc_readout(h, graph_ids, zeros))

    G = jnp.stack(g_parts)
    A = jnp.pad(x_adduct, ((0, 0), (0, D - 8)))
    W1g = W1[: 4 * D]
    W1a = jnp.pad(W1[4 * D:], ((0, D - 8), (0, 0)))
    Wop = jnp.pad(Wo, ((0, 0), (0, D - 1)))
    bop = jnp.pad(bo, (0, D - 1)).reshape(1, D)

    out = _head(G, A, W1g, W1a, b1.reshape(1, -1), W2, b2.reshape(1, -1),
                Wop, bop)
    return out[:, :1]


# speedup vs baseline: 10.0802x; 1.1333x over previous
import functools

import jax
import jax.numpy as jnp
from jax import lax
from jax.experimental import pallas as pl
from jax.experimental.pallas import tpu as pltpu
from jax.experimental.pallas import tpu_sc as plsc

N_NODES = 10000
N_EDGES = 320000
D = 128
N_GRAPHS = 256
NC = 2
NS = 16
NW = NC * NS
EPW = N_EDGES // NW
ECH = 80
NCHUNK = EPW // ECH
NBUF = 4
RCH = 80
NODE_CHUNKS = N_NODES // RCH

def _sc_agg_readout_body(h_hbm, src_hbm, dst_hbm, gid_hbm, zeros_hbm,
                         agg_out, g_out, *scr):
    idx_s = scr[0:NBUF]
    idx_d = scr[NBUF:2 * NBUF]
    rows = scr[2 * NBUF:3 * NBUF]
    seml = scr[3 * NBUF:4 * NBUF]
    semg = scr[4 * NBUF:5 * NBUF]
    sems = scr[5 * NBUF:6 * NBUF]
    acc, gacc = scr[6 * NBUF:]

    cid = lax.axis_index("c")
    sid = lax.axis_index("s")
    wid = sid * NC + cid

    @pl.when(sid == 0)
    def _init():
        pltpu.sync_copy(zeros_hbm, acc)
        pltpu.sync_copy(zeros_hbm.at[pl.ds(0, N_GRAPHS)], gacc)

    plsc.subcore_barrier()

    base = wid * EPW

    def start_l(c, b):
        off = base + c * ECH
        pltpu.async_copy(src_hbm.at[pl.ds(off, ECH)], idx_s[b], seml[b])
        pltpu.async_copy(dst_hbm.at[pl.ds(off, ECH)], idx_d[b], seml[b])

    def wait_l(b):
        pltpu.make_async_copy(src_hbm.at[pl.ds(0, ECH)], idx_s[b], seml[b]).wait()
        pltpu.make_async_copy(dst_hbm.at[pl.ds(0, ECH)], idx_d[b], seml[b]).wait()

    def start_g(b):
        pltpu.async_copy(h_hbm.at[idx_s[b]], rows[b], semg[b])

    def wait_g(b):
        pltpu.make_async_copy(h_hbm.at[idx_s[b]], rows[b], semg[b]).wait()

    def start_s(b):
        pltpu.async_copy(rows[b], acc.at[idx_d[b]], sems[b], add=True)

    def wait_s(b):
        pltpu.make_async_copy(rows[b], acc.at[idx_d[b]], sems[b]).wait()

    start_l(0, 0)
    start_l(1, 1)
    wait_l(0)
    start_g(0)

    def estep(i, carry):
        for b in range(NBUF):
            c = NBUF * i + b
            wait_l((b + 1) % NBUF)
            start_g((b + 1) % NBUF)
            wait_g(b)
            start_s(b)
            if b >= 2:
                wait_s((b + 2) % NBUF)
            else:
                @pl.when(i > 0)
                def _():
                    wait_s((b + 2) % NBUF)
            start_l(c + 2, (b + 2) % NBUF)
        return carry

    NMAIN = (NCHUNK - 2) // NBUF
    lax.fori_loop(0, NMAIN, estep, 0, unroll=False)

    for c in range(NMAIN * NBUF, NCHUNK):
        b = c % NBUF
        if c + 1 < NCHUNK:
            wait_l((b + 1) % NBUF)
            start_g((b + 1) % NBUF)
        wait_g(b)
        start_s(b)
        if c + 2 < NCHUNK:
            wait_s((b + 2) % NBUF)
            start_l(c + 2, (b + 2) % NBUF)

    for c in range(NCHUNK - NBUF, NCHUNK):
        wait_s(c % NBUF)

    def rstep(k, carry):
        c = wid + NW * k

        @pl.when(c < NODE_CHUNKS)
        def _():
            off = c * RCH
            pltpu.sync_copy(h_hbm.at[pl.ds(off, RCH)], rows[0])
            pltpu.sync_copy(gid_hbm.at[pl.ds(off, RCH)], idx_s[0])
            pltpu.sync_copy(rows[0], gacc.at[idx_s[0]], add=True)

        return carry

    lax.fori_loop(0, (NODE_CHUNKS + NW - 1) // NW, rstep, 0)

    plsc.subcore_barrier()

    @pl.when(sid == 0)
    def _writeout():
        pltpu.sync_copy(acc, agg_out.at[cid])
        pltpu.sync_copy(gacc, g_out.at[cid])


@functools.cache
def _sc_agg_readout_kernel():
    mesh = plsc.VectorSubcoreMesh(core_axis_name="c", subcore_axis_name="s")
    scratch = (
        [pltpu.VMEM((ECH,), jnp.int32)] * (2 * NBUF)
        + [pltpu.VMEM((ECH, D), jnp.float32)] * NBUF
        + [pltpu.SemaphoreType.DMA] * (3 * NBUF)
        + [
            pltpu.VMEM_SHARED((N_NODES, D), jnp.float32),
            pltpu.VMEM_SHARED((N_GRAPHS, D), jnp.float32),
        ]
    )
    return functools.partial(
        pl.kernel,
        out_type=(
            jax.ShapeDtypeStruct((NC, N_NODES, D), jnp.float32),
            jax.ShapeDtypeStruct((NC, N_GRAPHS, D), jnp.float32),
        ),
        mesh=mesh,
        scratch_types=scratch,
    )(_sc_agg_readout_body)


def _sc_agg_readout(h, src, dst, gid, zeros):
    return _sc_agg_readout_kernel()(h, src, dst, gid, zeros)


def _sc_readout_body(h_hbm, gid_hbm, zeros_hbm, g_out,
                     idx_a, rows, gacc):
    cid = lax.axis_index("c")
    sid = lax.axis_index("s")
    wid = sid * NC + cid

    @pl.when(sid == 0)
    def _init():
        pltpu.sync_copy(zeros_hbm.at[pl.ds(0, N_GRAPHS)], gacc)

    plsc.subcore_barrier()

    def rstep(k, carry):
        c = wid + NW * k

        @pl.when(c < NODE_CHUNKS)
        def _():
            off = c * RCH
            pltpu.sync_copy(h_hbm.at[pl.ds(off, RCH)], rows)
            pltpu.sync_copy(gid_hbm.at[pl.ds(off, RCH)], idx_a)
            pltpu.sync_copy(rows, gacc.at[idx_a], add=True)

        return carry

    lax.fori_loop(0, (NODE_CHUNKS + NW - 1) // NW, rstep, 0)

    plsc.subcore_barrier()

    @pl.when(sid == 0)
    def _writeout():
        pltpu.sync_copy(gacc, g_out.at[cid])


@functools.cache
def _sc_readout_kernel():
    mesh = plsc.VectorSubcoreMesh(core_axis_name="c", subcore_axis_name="s")
    return functools.partial(
        pl.kernel,
        out_type=jax.ShapeDtypeStruct((NC, N_GRAPHS, D), jnp.float32),
        mesh=mesh,
        scratch_types=[
            pltpu.VMEM((RCH,), jnp.int32),
            pltpu.VMEM((RCH, D), jnp.float32),
            pltpu.VMEM_SHARED((N_GRAPHS, D), jnp.float32),
        ],
    )(_sc_readout_body)


def _sc_readout(h, gid, zeros):
    return _sc_readout_kernel()(h, gid, zeros)


_ROWS_BLK = 1000
_GRID = N_NODES // _ROWS_BLK


def _proj_body(x_ref, w_ref, b_ref, o_ref):
    o_ref[...] = (
        jnp.dot(x_ref[...], w_ref[...], preferred_element_type=jnp.float32)
        + b_ref[...]
    )


def _proj(x, W, b):
    return pl.pallas_call(
        _proj_body,
        grid=(_GRID,),
        in_specs=[
            pl.BlockSpec((_ROWS_BLK, D), lambda i: (i, 0)),
            pl.BlockSpec((D, D), lambda i: (0, 0)),
            pl.BlockSpec((1, D), lambda i: (0, 0)),
        ],
        out_specs=pl.BlockSpec((_ROWS_BLK, D), lambda i: (i, 0)),
        out_shape=jax.ShapeDtypeStruct((N_NODES, D), jnp.float32),
    )(x, W, b)


def _gin_body(h_ref, a0_ref, a1_ref, w_ref, b_ref, s_ref, o_ref):
    t = s_ref[...] * h_ref[...] + a0_ref[...] + a1_ref[...]
    o_ref[...] = jnp.maximum(
        jnp.dot(t, w_ref[...], preferred_element_type=jnp.float32) + b_ref[...],
        0.0,
    )


def _gin_combine(h, a0, a1, W, b, scale):
    return pl.pallas_call(
        _gin_body,
        grid=(_GRID,),
        in_specs=[
            pl.BlockSpec((_ROWS_BLK, D), lambda i: (i, 0)),
            pl.BlockSpec((_ROWS_BLK, D), lambda i: (i, 0)),
            pl.BlockSpec((_ROWS_BLK, D), lambda i: (i, 0)),
            pl.BlockSpec((D, D), lambda i: (0, 0)),
            pl.BlockSpec((1, D), lambda i: (0, 0)),
            pl.BlockSpec((1, D), lambda i: (0, 0)),
        ],
        out_specs=pl.BlockSpec((_ROWS_BLK, D), lambda i: (i, 0)),
        out_shape=jax.ShapeDtypeStruct((N_NODES, D), jnp.float32),
    )(h, a0, a1, W, b, scale)


def _head_body(g_ref, a_ref, w1g_ref, w1a_ref, b1_ref, w2_ref, b2_ref,
               wo_ref, bo_ref, o_ref):
    acc = (
        jnp.dot(a_ref[...], w1a_ref[...], preferred_element_type=jnp.float32)
        + b1_ref[...]
    )
    for i in range(4):
        gi = g_ref[i, 0] + g_ref[i, 1]
        acc = acc + jnp.dot(
            gi, w1g_ref[i * D:(i + 1) * D, :], preferred_element_type=jnp.float32
        )
    z = jnp.maximum(acc, 0.0)
    z = jnp.maximum(
        jnp.dot(z, w2_ref[...], preferred_element_type=jnp.float32) + b2_ref[...],
        0.0,
    )
    o_ref[...] = (
        jnp.dot(z, wo_ref[...], preferred_element_type=jnp.float32) + bo_ref[...]
    )


def _head(G, A, W1g, W1a, b1, W2, b2, Wo, bo):
    return pl.pallas_call(
        _head_body,
        out_shape=jax.ShapeDtypeStruct((N_GRAPHS, D), jnp.float32),
    )(G, A, W1g, W1a, b1, W2, b2, Wo, bo)


def kernel(x, edge_index, graph_ids, x_adduct, W_proj, b_proj, W_gin, b_gin,
           eps, W1, b1, W2, b2, Wo, bo):
    src = edge_index[0]
    dst = edge_index[1]
    zeros = jnp.zeros((N_NODES, D), jnp.float32)

    h = _proj(x, W_proj, b_proj.reshape(1, D))
    g_parts = []
    for i in range(3):
        agg_p, g_p = _sc_agg_readout(h, src, dst, graph_ids, zeros)
        g_parts.append(g_p)
        scale = jnp.full((1, D), 1.0 + eps[i], jnp.float32)
        h = _gin_combine(h, agg_p[0], agg_p[1], W_gin[i],
                         b_gin[i].reshape(1, D), scale)
    g_parts.append(_sc_readout(h, graph_ids, zeros))

    G = jnp.stack(g_parts)
    A = jnp.pad(x_adduct, ((0, 0), (0, D - 8)))
    W1g = W1[: 4 * D]
    W1a = jnp.pad(W1[4 * D:], ((0, D - 8), (0, 0)))
    Wop = jnp.pad(Wo, ((0, 0), (0, D - 1)))
    bop = jnp.pad(bo, (0, D - 1)).reshape(1, D)

    out = _head(G, A, W1g, W1a, b1.reshape(1, -1), W2, b2.reshape(1, -1),
                Wop, bop)
    return out[:, :1]
